# Initial kernel scaffold; baseline (speedup 1.0000x reference)
import jax
import jax.numpy as jnp
from jax.experimental import pallas as pl


def kernel(images, boxes, classes, anchors):
    raise NotImplementedError("write your pallas kernel here")


# speedup vs baseline: 15.3980x; 15.3980x over previous
import functools

import jax
import jax.numpy as jnp
from jax.experimental import pallas as pl
from jax.experimental.pallas import tpu as pltpu


def _tc_body(af_ref, gt_ref, cls_ref, tx_ref, ty_ref, tw_ref, th_ref, tcls_ref):
    ax = af_ref[0]
    ay = af_ref[1]
    aw = af_ref[2]
    ah = af_ref[3]
    ax2 = ax + aw
    ay2 = ay + ah
    area_a = aw * ah
    n = gt_ref.shape[2]

    zero = jnp.zeros_like(ax)

    def body(j, carry):
        best, bcx, bcy, bw, bh, bcls = carry
        gx = gt_ref[0, 0, j]
        gy = gt_ref[0, 1, j]
        gw = gt_ref[0, 2, j]
        gh = gt_ref[0, 3, j]
        gx2 = gx + gw
        gy2 = gy + gh
        area_g = gw * gh
        ltx = jnp.maximum(ax, gx)
        lty = jnp.maximum(ay, gy)
        rbx = jnp.minimum(ax2, gx2)
        rby = jnp.minimum(ay2, gy2)
        wi = jnp.maximum(rbx - ltx, 0.0)
        hi = jnp.maximum(rby - lty, 0.0)
        inter = wi * hi
        union = area_a + area_g - inter
        iou = jnp.where(union > 0.0, inter / jnp.maximum(union, 1e-8), 0.0)
        upd = iou > best
        best = jnp.where(upd, iou, best)
        bcx = jnp.where(upd, gx + gw / 2.0, bcx)
        bcy = jnp.where(upd, gy + gh / 2.0, bcy)
        bw = jnp.where(upd, gw, bw)
        bh = jnp.where(upd, gh, bh)
        bcls = jnp.where(upd, cls_ref[0, 0, j], bcls)
        return best, bcx, bcy, bw, bh, bcls

    init = (jnp.full_like(ax, -1.0), zero, zero, zero, zero, zero)
    best, bcx, bcy, bw, bh, bcls = jax.lax.fori_loop(0, n, body, init)

    acx = ax + aw / 2.0
    acy = ay + ah / 2.0
    tx = ((bcx - acx) / aw) / 0.1
    ty = ((bcy - acy) / ah) / 0.1
    tw = jnp.log(bw / aw) / 0.2
    th = jnp.log(bh / ah) / 0.2
    tcls = jnp.where(best < 0.4, -1.0, jnp.where(best < 0.5, -2.0, bcls))
    nan_any = (
        jnp.isnan(tx) | jnp.isnan(ty) | jnp.isnan(tw) | jnp.isnan(th) | jnp.isnan(tcls)
    )
    tx_ref[0] = jnp.where(nan_any, -2.0, tx)
    ty_ref[0] = jnp.where(nan_any, -2.0, ty)
    tw_ref[0] = jnp.where(nan_any, -2.0, tw)
    th_ref[0] = jnp.where(nan_any, -2.0, th)
    tcls_ref[0] = jnp.where(nan_any, -2.0, tcls)


@jax.jit
def _encode_tc(boxes, classes, anchors):
    m = anchors.shape[0]
    b, n = classes.shape
    lanes = 128
    rblk = 48
    chunk = rblk * lanes
    m_pad = ((m + chunk - 1) // chunk) * chunk
    rows = m_pad // lanes

    pad = jnp.broadcast_to(
        jnp.array([0.0, 0.0, 1.0, 1.0], jnp.float32), (m_pad - m, 4)
    )
    af = jnp.concatenate([anchors, pad], axis=0).T.reshape(4, rows, lanes)
    gt = boxes.transpose(0, 2, 1)

    out_sd = jax.ShapeDtypeStruct((b, rows, lanes), jnp.float32)
    outs = pl.pallas_call(
        _tc_body,
        grid=(b, rows // rblk),
        in_specs=[
            pl.BlockSpec((4, rblk, lanes), lambda i, j: (0, j, 0)),
            pl.BlockSpec((1, 4, n), lambda i, j: (i, 0, 0), memory_space=pltpu.SMEM),
            pl.BlockSpec((1, 1, n), lambda i, j: (i, 0, 0), memory_space=pltpu.SMEM),
        ],
        out_specs=[
            pl.BlockSpec((1, rblk, lanes), lambda i, j: (i, j, 0)) for _ in range(5)
        ],
        out_shape=[out_sd] * 5,
    )(af, gt, classes.reshape(b, 1, n))

    tx, ty, tw, th, tcls = outs
    box = jnp.stack([tx, ty, tw, th], axis=-1).reshape(b, m_pad, 4)[:, :m]
    return box, tcls.reshape(b, m_pad)[:, :m]


def kernel(images, boxes, classes, anchors):
    del images
    return _encode_tc(boxes, classes, anchors)


# speedup vs baseline: 17.7357x; 1.1518x over previous
import functools

import jax
import jax.numpy as jnp
from jax.experimental import pallas as pl
from jax.experimental.pallas import tpu as pltpu


def _tc_body(af_ref, gt_ref, cls_ref, tx_ref, ty_ref, tw_ref, th_ref, tcls_ref):
    ax = af_ref[0]
    ay = af_ref[1]
    aw = af_ref[2]
    ah = af_ref[3]
    ax2 = ax + aw
    ay2 = ay + ah
    area_a = aw * ah
    n = gt_ref.shape[2]

    zero = jnp.zeros_like(ax)

    def body(j, carry):
        best, bcx, bcy, bw, bh, bcls = carry
        gx = gt_ref[0, 0, j]
        gy = gt_ref[0, 1, j]
        gw = gt_ref[0, 2, j]
        gh = gt_ref[0, 3, j]
        gx2 = gx + gw
        gy2 = gy + gh
        area_g = gw * gh
        ltx = jnp.maximum(ax, gx)
        lty = jnp.maximum(ay, gy)
        rbx = jnp.minimum(ax2, gx2)
        rby = jnp.minimum(ay2, gy2)
        wi = jnp.maximum(rbx - ltx, 0.0)
        hi = jnp.maximum(rby - lty, 0.0)
        inter = wi * hi
        union = area_a + area_g - inter
        iou = inter / union
        upd = iou > best
        best = jnp.where(upd, iou, best)
        bcx = jnp.where(upd, gx + gw / 2.0, bcx)
        bcy = jnp.where(upd, gy + gh / 2.0, bcy)
        bw = jnp.where(upd, gw, bw)
        bh = jnp.where(upd, gh, bh)
        bcls = jnp.where(upd, cls_ref[0, 0, j], bcls)
        return best, bcx, bcy, bw, bh, bcls

    init = (jnp.full_like(ax, -1.0), zero, zero, zero, zero, zero)
    best, bcx, bcy, bw, bh, bcls = jax.lax.fori_loop(0, n, body, init)

    acx = ax + aw / 2.0
    acy = ay + ah / 2.0
    tx = ((bcx - acx) / aw) / 0.1
    ty = ((bcy - acy) / ah) / 0.1
    tw = jnp.log(bw / aw) / 0.2
    th = jnp.log(bh / ah) / 0.2
    tcls = jnp.where(best < 0.4, -1.0, jnp.where(best < 0.5, -2.0, bcls))
    nan_any = (
        jnp.isnan(tx) | jnp.isnan(ty) | jnp.isnan(tw) | jnp.isnan(th) | jnp.isnan(tcls)
    )
    tx_ref[0] = jnp.where(nan_any, -2.0, tx)
    ty_ref[0] = jnp.where(nan_any, -2.0, ty)
    tw_ref[0] = jnp.where(nan_any, -2.0, tw)
    th_ref[0] = jnp.where(nan_any, -2.0, th)
    tcls_ref[0] = jnp.where(nan_any, -2.0, tcls)


@jax.jit
def _encode_tc(boxes, classes, anchors):
    m = anchors.shape[0]
    b, n = classes.shape
    lanes = 128
    rblk = 96
    chunk = rblk * lanes
    m_pad = ((m + chunk - 1) // chunk) * chunk
    rows = m_pad // lanes

    pad = jnp.broadcast_to(
        jnp.array([0.0, 0.0, 1.0, 1.0], jnp.float32), (m_pad - m, 4)
    )
    af = jnp.concatenate([anchors, pad], axis=0).T.reshape(4, rows, lanes)
    gt = boxes.transpose(0, 2, 1)

    out_sd = jax.ShapeDtypeStruct((b, rows, lanes), jnp.float32)
    outs = pl.pallas_call(
        _tc_body,
        grid=(b, rows // rblk),
        in_specs=[
            pl.BlockSpec((4, rblk, lanes), lambda i, j: (0, j, 0)),
            pl.BlockSpec((1, 4, n), lambda i, j: (i, 0, 0), memory_space=pltpu.SMEM),
            pl.BlockSpec((1, 1, n), lambda i, j: (i, 0, 0), memory_space=pltpu.SMEM),
        ],
        out_specs=[
            pl.BlockSpec((1, rblk, lanes), lambda i, j: (i, j, 0)) for _ in range(5)
        ],
        out_shape=[out_sd] * 5,
    )(af, gt, classes.reshape(b, 1, n))

    tx, ty, tw, th, tcls = outs
    box = jnp.stack([tx, ty, tw, th], axis=-1).reshape(b, m_pad, 4)[:, :m]
    return box, tcls.reshape(b, m_pad)[:, :m]


def kernel(images, boxes, classes, anchors):
    del images
    return _encode_tc(boxes, classes, anchors)
